# baseline (speedup 1.0000x reference)
import numpy as np
import jax
import jax.numpy as jnp
from jax import lax
from jax.experimental import pallas as pl
from jax.experimental.pallas import tpu as pltpu
from jax.experimental.pallas import tpu_sc as plsc

_N = 10000
_E = 320000
_D = 128
_H = 128
_B = 2

_NC = 2
_NS = 16
_NW = _NC * _NS
_CH = 128

_f32 = jnp.float32


def _build_consts():
    g1 = np.zeros((16, 48), np.float32)
    g2 = np.zeros((16, 48), np.float32)
    g3 = np.zeros((48, 16), np.float32)
    g4 = np.zeros((4, 16), np.float32)
    for j in range(4):
        for k in range(4):
            for t in range(3):
                q = 3 * (4 * j + k) + t
                g1[3 * j + t, q] = 1.0
                g2[3 * k + t, q] = 1.0
                g3[q, 4 * j + k] = 1.0
    for j in range(4):
        for t in range(3):
            g4[j, 3 * j + t] = 1.0
    p12 = np.eye(12, 16, dtype=np.float32)
    oh12 = np.zeros((1, 16), np.float32)
    oh12[0, 12] = 1.0
    p128 = np.eye(12, 128, dtype=np.float32)
    oh128 = np.zeros((1, 128), np.float32)
    oh128[0, 12] = 1.0
    sel = np.eye(128, 16, dtype=np.float32)
    return g1, g2, g3, g4, p12, p128, oh12, oh128, sel


_G1, _G2, _G3, _G4, _P12, _P128, _OH12, _OH128, _SEL = _build_consts()


def _pre_body(h_ref, c12_ref, ws_ref, wt_ref, p12_ref, p128_ref,
              hws_ref, hwt_ref, c16_ref, ct_ref):
    hb = h_ref[...]
    hws_ref[...] = jnp.dot(hb, ws_ref[...], preferred_element_type=_f32)
    hwt_ref[...] = jnp.dot(hb, wt_ref[...], preferred_element_type=_f32)
    c12 = c12_ref[...]
    c16_ref[...] = jnp.dot(c12, p12_ref[...], preferred_element_type=_f32)
    ct_ref[...] = jnp.dot(c12, p128_ref[...], preferred_element_type=_f32)


def _pre_call_kwargs():
    kb = 2000
    r = _B * _N
    return dict(
        grid=(r // kb,),
        in_specs=[
            pl.BlockSpec((kb, _D), lambda i: (i, 0)),
            pl.BlockSpec((kb, 12), lambda i: (i, 0)),
            pl.BlockSpec((_D, _H), lambda i: (0, 0)),
            pl.BlockSpec((_D, _H), lambda i: (0, 0)),
            pl.BlockSpec((12, 16), lambda i: (0, 0)),
            pl.BlockSpec((12, _D), lambda i: (0, 0)),
        ],
        out_specs=[
            pl.BlockSpec((kb, _H), lambda i: (i, 0)),
            pl.BlockSpec((kb, _H), lambda i: (i, 0)),
            pl.BlockSpec((kb, 16), lambda i: (i, 0)),
            pl.BlockSpec((kb, _D), lambda i: (i, 0)),
        ],
        out_shape=[
            jax.ShapeDtypeStruct((r, _H), _f32),
            jax.ShapeDtypeStruct((r, _H), _f32),
            jax.ShapeDtypeStruct((r, 16), _f32),
            jax.ShapeDtypeStruct((r, _D), _f32),
        ],
    )


_TSTART = 624
_TSIZE = 640


_EPW = _E // _NW
_CHG = 64
_NKG = (_EPW + _CHG - 1) // _CHG
_LASTBASE = _EPW - _CHG


def _gather_body(row_hbm, col_hbm, hws0, hwt0, ct0,
                 src0, tgt0, cdf0,
                 idxr, idxc, rs0, rs1, rt0, rt1, cbr0, cbr1, cbc0, cbc1,
                 cdp0, cdp1,
                 sr0, sr1, st0, st1, scr0, scr1, scc0, scc1):
    s = lax.axis_index("s")
    wid = s * _NC + lax.axis_index("c")
    wbase = wid * _EPW
    RS = (rs0, rs1)
    RT = (rt0, rt1)
    CBR = (cbr0, cbr1)
    CBC = (cbc0, cbc1)
    CDP = (cdp0, cdp1)
    SR = (sr0, sr1)
    ST = (st0, st1)
    SCR = (scr0, scr1)
    SCC = (scc0, scc1)
    for hws, hwt, ct, so, to, cdo in (
            (hws0, hwt0, ct0, src0, tgt0, cdf0),):
        pltpu.sync_copy(row_hbm.at[pl.ds(wbase, _EPW)], idxr)
        pltpu.sync_copy(col_hbm.at[pl.ds(wbase, _EPW)], idxc)

        def issue(k, p):
            @pl.when(k < _NKG)
            def _():
                bl = jnp.minimum(k * _CHG, _LASTBASE)
                pltpu.async_copy(hws.at[idxr.at[pl.ds(bl, _CHG)]],
                                 RS[p], SR[p])
                pltpu.async_copy(hwt.at[idxc.at[pl.ds(bl, _CHG)]],
                                 RT[p], ST[p])
                pltpu.async_copy(ct.at[idxr.at[pl.ds(bl, _CHG)]],
                                 CBR[p], SCR[p])
                pltpu.async_copy(ct.at[idxc.at[pl.ds(bl, _CHG)]],
                                 CBC[p], SCC[p])

        def drain(k, p):
            @pl.when(k < _NKG)
            def _():
                bl = jnp.minimum(k * _CHG, _LASTBASE)
                base = wbase + bl
                pltpu.make_async_copy(hws.at[idxr.at[pl.ds(bl, _CHG)]],
                                      RS[p], SR[p]).wait()
                pltpu.sync_copy(RS[p], so.at[pl.ds(base, _CHG)])
                pltpu.make_async_copy(hwt.at[idxc.at[pl.ds(bl, _CHG)]],
                                      RT[p], ST[p]).wait()
                pltpu.sync_copy(RT[p], to.at[pl.ds(base, _CHG)])
                pltpu.make_async_copy(ct.at[idxr.at[pl.ds(bl, _CHG)]],
                                      CBR[p], SCR[p]).wait()
                pltpu.make_async_copy(ct.at[idxc.at[pl.ds(bl, _CHG)]],
                                      CBC[p], SCC[p]).wait()

                def pack(e, carry2):
                    CDP[p][pl.ds(e * 16, 16)] = (CBR[p][e, pl.ds(0, 16)]
                                                 - CBC[p][e, pl.ds(0, 16)])
                    return carry2
                lax.fori_loop(0, _CHG, pack, 0)
                pltpu.sync_copy(CDP[p], cdo.at[pl.ds(base * 16, _CHG * 16)])

        issue(0, 0)

        def pipe(j2, carry):
            issue(2 * j2 + 1, 1)
            drain(2 * j2, 0)
            issue(2 * j2 + 2, 0)
            drain(2 * j2 + 1, 1)
            return carry
        lax.fori_loop(0, (_NKG + 1) // 2, pipe, 0)


def _edge_body(src_ref, tgt_ref, cd_ref, ea_ref,
               wr_ref, wa_ref, be1_ref, we2_ref, be2_ref,
               wc1_ref, bc1_ref, wcg_ref, g1_ref, g2_ref, g3_ref, oh_ref,
               ef_ref, tr_ref):
    cd16 = cd_ref[...]
    x1 = jnp.dot(cd16, g1_ref[...], preferred_element_type=_f32)
    x2 = jnp.dot(cd16, g2_ref[...], preferred_element_type=_f32)
    pro = jnp.dot(x1 * x2, g3_ref[...], preferred_element_type=_f32)
    ss = jnp.sum(pro * pro, axis=1, keepdims=True)
    radial = pro / jnp.maximum(jnp.sqrt(ss), 1e-12)
    e1 = (src_ref[...] + tgt_ref[...]
          + jnp.dot(radial, wr_ref[...], preferred_element_type=_f32)
          + jnp.dot(ea_ref[...], wa_ref[...], preferred_element_type=_f32)
          + be1_ref[...])
    e1 = jnp.maximum(e1, 0.0)
    ef = jnp.maximum(
        jnp.dot(e1, we2_ref[...], preferred_element_type=_f32) + be2_ref[...],
        0.0)
    ef_ref[...] = ef
    mm = jnp.maximum(
        jnp.dot(ef, wc1_ref[...], preferred_element_type=_f32) + bc1_ref[...],
        0.0)
    m16 = jnp.dot(mm, wcg_ref[...], preferred_element_type=_f32)
    tr_ref[...] = cd16 * m16 + oh_ref[...]


def _edge_call_kwargs():
    ke = 2000
    return dict(
        grid=(_E // ke,),
        in_specs=[
            pl.BlockSpec((ke, _D), lambda i: (i, 0)),
            pl.BlockSpec((ke, _D), lambda i: (i, 0)),
            pl.BlockSpec((ke, 16), lambda i: (i, 0)),
            pl.BlockSpec((ke, 16), lambda i: (i, 0)),
            pl.BlockSpec((16, _H), lambda i: (0, 0)),
            pl.BlockSpec((16, _H), lambda i: (0, 0)),
            pl.BlockSpec((1, _H), lambda i: (0, 0)),
            pl.BlockSpec((_H, _H), lambda i: (0, 0)),
            pl.BlockSpec((1, _H), lambda i: (0, 0)),
            pl.BlockSpec((_H, _H), lambda i: (0, 0)),
            pl.BlockSpec((1, _H), lambda i: (0, 0)),
            pl.BlockSpec((_H, 16), lambda i: (0, 0)),
            pl.BlockSpec((16, 48), lambda i: (0, 0)),
            pl.BlockSpec((16, 48), lambda i: (0, 0)),
            pl.BlockSpec((48, 16), lambda i: (0, 0)),
            pl.BlockSpec((1, 16), lambda i: (0, 0)),
        ],
        out_specs=[
            pl.BlockSpec((ke, _D), lambda i: (i, 0)),
            pl.BlockSpec((ke, 16), lambda i: (i, 0)),
        ],
        out_shape=[
            jax.ShapeDtypeStruct((_E, _D), _f32),
            jax.ShapeDtypeStruct((_E, 16), _f32),
        ],
    )


_CHS = 80
_NKS = _EPW // _CHS


def _scatter_body(row_hbm, ef0, trf0, zh,
                  aggp0, acp0,
                  ix0, ix1, rv0, rv1, tv0, tv1, big, sh_acc,
                  si0, si1, sd0, sd1):
    c = lax.axis_index("c")
    s = lax.axis_index("s")
    wid = s * _NC + c
    wbase = wid * _EPW
    IX = (ix0, ix1)
    RV = (rv0, rv1)
    TV = (tv0, tv1)
    SI = (si0, si1)
    SD = (sd0, sd1)
    z16 = jnp.zeros((16,), _f32)
    pltpu.sync_copy(zh, big)
    for ef, trf, aggp, acp in ((ef0, trf0, aggp0, acp0),):
        def clr(e, carry):
            big[e, pl.ds(0, 16)] = z16
            return carry
        lax.fori_loop(0, _CHS, clr, 0)

        for pass_b in (False, True):
            for i in range(_TSIZE // _CH):
                pltpu.sync_copy(big,
                                sh_acc.at[pl.ds(s * _TSTART + i * _CH, _CH)])
            plsc.subcore_barrier()

            def issue(k, p):
                @pl.when(k < _NKS)
                def _():
                    base = wbase + k * _CHS
                    pltpu.async_copy(row_hbm.at[pl.ds(base, _CHS)],
                                     IX[p], SI[p])
                    if pass_b:
                        pltpu.async_copy(trf.at[pl.ds(base * 16, _CHS * 16)],
                                         TV[p], SD[p])
                    else:
                        pltpu.async_copy(ef.at[pl.ds(base, _CHS)],
                                         RV[p], SD[p])

            def drain(k, p):
                @pl.when(k < _NKS)
                def _():
                    base = wbase + k * _CHS
                    pltpu.make_async_copy(row_hbm.at[pl.ds(base, _CHS)],
                                          IX[p], SI[p]).wait()
                    if pass_b:
                        pltpu.make_async_copy(
                            trf.at[pl.ds(base * 16, _CHS * 16)],
                            TV[p], SD[p]).wait()

                        def expand(e, carry2):
                            big[e, pl.ds(0, 16)] = TV[p][pl.ds(e * 16, 16)]
                            return carry2
                        lax.fori_loop(0, _CHS, expand, 0)
                        pltpu.sync_copy(big.at[pl.ds(0, _CHS)],
                                        sh_acc.at[IX[p]], add=True)
                    else:
                        pltpu.make_async_copy(ef.at[pl.ds(base, _CHS)],
                                              RV[p], SD[p]).wait()
                        pltpu.sync_copy(RV[p], sh_acc.at[IX[p]], add=True)

            issue(0, 0)

            def pipe(j2, carry):
                issue(2 * j2 + 1, 1)
                drain(2 * j2, 0)
                issue(2 * j2 + 2, 0)
                drain(2 * j2 + 1, 1)
                return carry
            lax.fori_loop(0, (_NKS + 1) // 2, pipe, 0)
            plsc.subcore_barrier()
            out = acp if pass_b else aggp
            pltpu.sync_copy(sh_acc.at[pl.ds(s * _TSTART, _TSIZE)],
                            out.at[pl.ds(c * _N + s * _TSTART, _TSIZE)])
            plsc.subcore_barrier()
            if pass_b:
                pass


def _node_body(h_ref, c16_ref, ap0_ref, ap1_ref, ac0_ref, ac1_ref,
               wnh_ref, wna_ref, bn1_ref, wn2_ref, bn2_ref, oh_ref, sel_ref,
               hout_ref, cout_ref):
    agg = ap0_ref[...] + ap1_ref[...]
    ac = ac0_ref[...] + ac1_ref[...]
    deg = jnp.sum(ac * oh_ref[...], axis=1, keepdims=True)
    ac16 = jnp.dot(ac, sel_ref[...], preferred_element_type=_f32)
    cout_ref[...] = c16_ref[...] + ac16 * (1.0 / jnp.maximum(deg, 1.0))
    hb = h_ref[...]
    t = jnp.maximum(
        jnp.dot(hb, wnh_ref[...], preferred_element_type=_f32)
        + jnp.dot(agg, wna_ref[...], preferred_element_type=_f32)
        + bn1_ref[...], 0.0)
    hout_ref[...] = (hb + jnp.dot(t, wn2_ref[...], preferred_element_type=_f32)
                     + bn2_ref[...])


def _node_call_kwargs():
    kn = 2000
    return dict(
        grid=(_N // kn,),
        in_specs=[
            pl.BlockSpec((kn, _D), lambda i: (i, 0)),
            pl.BlockSpec((kn, 16), lambda i: (i, 0)),
            pl.BlockSpec((kn, _D), lambda i: (i, 0)),
            pl.BlockSpec((kn, _D), lambda i: (i, 0)),
            pl.BlockSpec((kn, _D), lambda i: (i, 0)),
            pl.BlockSpec((kn, _D), lambda i: (i, 0)),
            pl.BlockSpec((_D, _H), lambda i: (0, 0)),
            pl.BlockSpec((_H, _H), lambda i: (0, 0)),
            pl.BlockSpec((1, _H), lambda i: (0, 0)),
            pl.BlockSpec((_H, _D), lambda i: (0, 0)),
            pl.BlockSpec((1, _D), lambda i: (0, 0)),
            pl.BlockSpec((1, _D), lambda i: (0, 0)),
            pl.BlockSpec((_D, 16), lambda i: (0, 0)),
        ],
        out_specs=[
            pl.BlockSpec((kn, _D), lambda i: (i, 0)),
            pl.BlockSpec((kn, 16), lambda i: (i, 0)),
        ],
        out_shape=[
            jax.ShapeDtypeStruct((_N, _D), _f32),
            jax.ShapeDtypeStruct((_N, 16), _f32),
        ],
    )


def kernel(h, coord, edge_index, edge_attr, We1, be1, We2, be2,
           Wn1, bn1, Wn2, bn2, Wc1, bc1, Wc2):
    row = edge_index[0]
    col = edge_index[1]
    hflat = h.reshape(_B * _N, _D)
    c12 = coord.reshape(_B * _N, 12)

    ws = We1[:_D]
    wt = We1[_D:2 * _D]
    wr = We1[2 * _D:2 * _D + 16]
    wa = We1[2 * _D + 16:]
    wcg = Wc2 @ jnp.asarray(_G4)
    g1 = jnp.asarray(_G1)
    g2 = jnp.asarray(_G2)
    g3 = jnp.asarray(_G3)
    p12 = jnp.asarray(_P12)
    p128 = jnp.asarray(_P128)
    oh12 = jnp.asarray(_OH12)
    oh128 = jnp.asarray(_OH128)
    sel = jnp.asarray(_SEL)
    be1r = be1.reshape(1, _H)
    be2r = be2.reshape(1, _H)
    bc1r = bc1.reshape(1, _H)
    bn1r = bn1.reshape(1, _H)
    bn2r = bn2.reshape(1, _D)
    wnh = Wn1[:_D]
    wna = Wn1[_D:]

    hws, hwt, c16, ct = pl.pallas_call(_pre_body, **_pre_call_kwargs())(
        hflat, c12, ws, wt, p12, p128)

    gather = pl.kernel(
        _gather_body,
        out_type=(
            jax.ShapeDtypeStruct((_E, _D), _f32),
            jax.ShapeDtypeStruct((_E, _D), _f32),
            jax.ShapeDtypeStruct((_E * 16,), _f32),
        ),
        mesh=plsc.VectorSubcoreMesh(core_axis_name="c", subcore_axis_name="s"),
        scratch_types=(
            [pltpu.VMEM((_EPW,), jnp.int32)] * 2
            + [pltpu.VMEM((_CHG, _D), _f32)] * 8
            + [pltpu.VMEM((_CHG * 16,), _f32)] * 2
            + [pltpu.SemaphoreType.DMA] * 8
        ),
    )
    src0, tgt0, cdf0 = gather(row, col, hws[:_N], hwt[:_N], ct[:_N])
    src1, tgt1, cdf1 = gather(row, col, hws[_N:], hwt[_N:], ct[_N:])
    cd0 = cdf0.reshape(_E, 16)
    cd1 = cdf1.reshape(_E, 16)

    edge_call = pl.pallas_call(_edge_body, **_edge_call_kwargs())
    ef0, tr0 = edge_call(src0, tgt0, cd0, edge_attr, wr, wa, be1r,
                         We2, be2r, Wc1, bc1r, wcg, g1, g2, g3, oh12)
    ef1, tr1 = edge_call(src1, tgt1, cd1, edge_attr, wr, wa, be1r,
                         We2, be2r, Wc1, bc1r, wcg, g1, g2, g3, oh12)

    zh = jnp.zeros((_CH, _D), _f32)
    scatter = pl.kernel(
        _scatter_body,
        out_type=tuple(
            jax.ShapeDtypeStruct((_NC * _N, _D), _f32) for _ in range(2)),
        mesh=plsc.VectorSubcoreMesh(core_axis_name="c", subcore_axis_name="s"),
        scratch_types=(
            [pltpu.VMEM((_CHS,), jnp.int32)] * 2
            + [pltpu.VMEM((_CHS, _D), _f32)] * 2
            + [pltpu.VMEM((_CHS * 16,), _f32)] * 2
            + [pltpu.VMEM((_CH, _D), _f32),
               pltpu.VMEM_SHARED((_N, _D), _f32)]
            + [pltpu.SemaphoreType.DMA] * 4
        ),
    )
    aggp0, acp0 = scatter(row, ef0, tr0.reshape(-1), zh)
    aggp1, acp1 = scatter(row, ef1, tr1.reshape(-1), zh)
    node_call = pl.pallas_call(_node_body, **_node_call_kwargs())
    h0, c0 = node_call(hflat[:_N], c16[:_N], aggp0[:_N], aggp0[_N:],
                       acp0[:_N], acp0[_N:], wnh, wna, bn1r, Wn2, bn2r,
                       oh128, sel)
    h1, c1 = node_call(hflat[_N:], c16[_N:], aggp1[:_N], aggp1[_N:],
                       acp1[:_N], acp1[_N:], wnh, wna, bn1r, Wn2, bn2r,
                       oh128, sel)

    hs = jnp.stack([h0, h1])
    coords = jnp.stack([c0[:, :12].reshape(_N, 4, 3),
                        c1[:, :12].reshape(_N, 4, 3)])
    return (hs, coords)


# speedup vs baseline: 1.0939x; 1.0939x over previous
import numpy as np
import jax
import jax.numpy as jnp
from jax import lax
from jax.experimental import pallas as pl
from jax.experimental.pallas import tpu as pltpu
from jax.experimental.pallas import tpu_sc as plsc

_N = 10000
_E = 320000
_D = 128
_H = 128
_B = 2

_NC = 2
_NS = 16
_NW = _NC * _NS
_CH = 128

_f32 = jnp.float32


def _build_consts():
    g1 = np.zeros((16, 48), np.float32)
    g2 = np.zeros((16, 48), np.float32)
    g3 = np.zeros((48, 16), np.float32)
    g4 = np.zeros((4, 16), np.float32)
    for j in range(4):
        for k in range(4):
            for t in range(3):
                q = 3 * (4 * j + k) + t
                g1[3 * j + t, q] = 1.0
                g2[3 * k + t, q] = 1.0
                g3[q, 4 * j + k] = 1.0
    for j in range(4):
        for t in range(3):
            g4[j, 3 * j + t] = 1.0
    p12 = np.eye(12, 16, dtype=np.float32)
    oh12 = np.zeros((1, 16), np.float32)
    oh12[0, 12] = 1.0
    p128 = np.eye(12, 128, dtype=np.float32)
    oh128 = np.zeros((1, 128), np.float32)
    oh128[0, 12] = 1.0
    sel = np.eye(128, 16, dtype=np.float32)
    return g1, g2, g3, g4, p12, p128, oh12, oh128, sel


_G1, _G2, _G3, _G4, _P12, _P128, _OH12, _OH128, _SEL = _build_consts()


def _pre_body(h_ref, c12_ref, ws_ref, wt_ref, p12_ref, p128_ref,
              hws_ref, hwt_ref, c16_ref, ct_ref):
    hb = h_ref[...]
    hws_ref[...] = jnp.dot(hb, ws_ref[...], preferred_element_type=_f32)
    hwt_ref[...] = jnp.dot(hb, wt_ref[...], preferred_element_type=_f32)
    c12 = c12_ref[...]
    c16_ref[...] = jnp.dot(c12, p12_ref[...], preferred_element_type=_f32)
    ct_ref[...] = jnp.dot(c12, p128_ref[...], preferred_element_type=_f32)


def _pre_call_kwargs():
    kb = 2000
    r = _B * _N
    return dict(
        grid=(r // kb,),
        in_specs=[
            pl.BlockSpec((kb, _D), lambda i: (i, 0)),
            pl.BlockSpec((kb, 12), lambda i: (i, 0)),
            pl.BlockSpec((_D, _H), lambda i: (0, 0)),
            pl.BlockSpec((_D, _H), lambda i: (0, 0)),
            pl.BlockSpec((12, 16), lambda i: (0, 0)),
            pl.BlockSpec((12, _D), lambda i: (0, 0)),
        ],
        out_specs=[
            pl.BlockSpec((kb, _H), lambda i: (i, 0)),
            pl.BlockSpec((kb, _H), lambda i: (i, 0)),
            pl.BlockSpec((kb, 16), lambda i: (i, 0)),
            pl.BlockSpec((kb, _D), lambda i: (i, 0)),
        ],
        out_shape=[
            jax.ShapeDtypeStruct((r, _H), _f32),
            jax.ShapeDtypeStruct((r, _H), _f32),
            jax.ShapeDtypeStruct((r, 16), _f32),
            jax.ShapeDtypeStruct((r, _D), _f32),
        ],
    )


_TSTART = 624
_TSIZE = 640


_EPW = _E // _NW
_CHG = 64
_NKG = (_EPW + _CHG - 1) // _CHG
_LASTBASE = _EPW - _CHG


def _gather_body(row_hbm, col_hbm, hws0, hwt0, ct0,
                 pre0, cdf0,
                 idxr, idxc, rs0, rs1, rt0, rt1, cbr0, cbr1, cbc0, cbc1,
                 cdp0, cdp1,
                 sr0, sr1, st0, st1, scr0, scr1, scc0, scc1):
    s = lax.axis_index("s")
    wid = s * _NC + lax.axis_index("c")
    wbase = wid * _EPW
    RS = (rs0, rs1)
    RT = (rt0, rt1)
    CBR = (cbr0, cbr1)
    CBC = (cbc0, cbc1)
    CDP = (cdp0, cdp1)
    SR = (sr0, sr1)
    ST = (st0, st1)
    SCR = (scr0, scr1)
    SCC = (scc0, scc1)
    for hws, hwt, ct, po, cdo in (
            (hws0, hwt0, ct0, pre0, cdf0),):
        pltpu.sync_copy(row_hbm.at[pl.ds(wbase, _EPW)], idxr)
        pltpu.sync_copy(col_hbm.at[pl.ds(wbase, _EPW)], idxc)

        def issue(k, p):
            @pl.when(k < _NKG)
            def _():
                bl = jnp.minimum(k * _CHG, _LASTBASE)
                pltpu.async_copy(hws.at[idxr.at[pl.ds(bl, _CHG)]],
                                 RS[p], SR[p])
                pltpu.async_copy(hwt.at[idxc.at[pl.ds(bl, _CHG)]],
                                 RT[p], ST[p])
                pltpu.async_copy(ct.at[idxr.at[pl.ds(bl, _CHG)]],
                                 CBR[p], SCR[p])
                pltpu.async_copy(ct.at[idxc.at[pl.ds(bl, _CHG)]],
                                 CBC[p], SCC[p])

        def drain(k, p):
            @pl.when(k < _NKG)
            def _():
                bl = jnp.minimum(k * _CHG, _LASTBASE)
                base = wbase + bl
                pltpu.make_async_copy(hws.at[idxr.at[pl.ds(bl, _CHG)]],
                                      RS[p], SR[p]).wait()
                pltpu.make_async_copy(hwt.at[idxc.at[pl.ds(bl, _CHG)]],
                                      RT[p], ST[p]).wait()

                def addrow(e, carry2):
                    for g in range(8):
                        RS[p][e, pl.ds(g * 16, 16)] = (
                            RS[p][e, pl.ds(g * 16, 16)]
                            + RT[p][e, pl.ds(g * 16, 16)])
                    return carry2
                lax.fori_loop(0, _CHG, addrow, 0)
                pltpu.sync_copy(RS[p], po.at[pl.ds(base, _CHG)])
                pltpu.make_async_copy(ct.at[idxr.at[pl.ds(bl, _CHG)]],
                                      CBR[p], SCR[p]).wait()
                pltpu.make_async_copy(ct.at[idxc.at[pl.ds(bl, _CHG)]],
                                      CBC[p], SCC[p]).wait()

                def pack(e, carry2):
                    CDP[p][pl.ds(e * 16, 16)] = (CBR[p][e, pl.ds(0, 16)]
                                                 - CBC[p][e, pl.ds(0, 16)])
                    return carry2
                lax.fori_loop(0, _CHG, pack, 0)
                pltpu.sync_copy(CDP[p], cdo.at[pl.ds(base * 16, _CHG * 16)])

        issue(0, 0)

        def pipe(j2, carry):
            issue(2 * j2 + 1, 1)
            drain(2 * j2, 0)
            issue(2 * j2 + 2, 0)
            drain(2 * j2 + 1, 1)
            return carry
        lax.fori_loop(0, (_NKG + 1) // 2, pipe, 0)


def _edge_body(pre_ref, cd_ref, ea_ref,
               wr_ref, wa_ref, be1_ref, we2_ref, be2_ref,
               wc1_ref, bc1_ref, wcg_ref, g1_ref, g2_ref, g3_ref, oh_ref,
               ef_ref, tr_ref):
    cd16 = cd_ref[...]
    x1 = jnp.dot(cd16, g1_ref[...], preferred_element_type=_f32)
    x2 = jnp.dot(cd16, g2_ref[...], preferred_element_type=_f32)
    pro = jnp.dot(x1 * x2, g3_ref[...], preferred_element_type=_f32)
    ss = jnp.sum(pro * pro, axis=1, keepdims=True)
    radial = pro / jnp.maximum(jnp.sqrt(ss), 1e-12)
    e1 = (pre_ref[...]
          + jnp.dot(radial, wr_ref[...], preferred_element_type=_f32)
          + jnp.dot(ea_ref[...], wa_ref[...], preferred_element_type=_f32)
          + be1_ref[...])
    e1 = jnp.maximum(e1, 0.0)
    ef = jnp.maximum(
        jnp.dot(e1, we2_ref[...], preferred_element_type=_f32) + be2_ref[...],
        0.0)
    ef_ref[...] = ef
    mm = jnp.maximum(
        jnp.dot(ef, wc1_ref[...], preferred_element_type=_f32) + bc1_ref[...],
        0.0)
    m16 = jnp.dot(mm, wcg_ref[...], preferred_element_type=_f32)
    tr_ref[...] = cd16 * m16 + oh_ref[...]


def _edge_call_kwargs():
    ke = 2000
    return dict(
        grid=(_E // ke,),
        in_specs=[
            pl.BlockSpec((ke, _D), lambda i: (i, 0)),
            pl.BlockSpec((ke, 16), lambda i: (i, 0)),
            pl.BlockSpec((ke, 16), lambda i: (i, 0)),
            pl.BlockSpec((16, _H), lambda i: (0, 0)),
            pl.BlockSpec((16, _H), lambda i: (0, 0)),
            pl.BlockSpec((1, _H), lambda i: (0, 0)),
            pl.BlockSpec((_H, _H), lambda i: (0, 0)),
            pl.BlockSpec((1, _H), lambda i: (0, 0)),
            pl.BlockSpec((_H, _H), lambda i: (0, 0)),
            pl.BlockSpec((1, _H), lambda i: (0, 0)),
            pl.BlockSpec((_H, 16), lambda i: (0, 0)),
            pl.BlockSpec((16, 48), lambda i: (0, 0)),
            pl.BlockSpec((16, 48), lambda i: (0, 0)),
            pl.BlockSpec((48, 16), lambda i: (0, 0)),
            pl.BlockSpec((1, 16), lambda i: (0, 0)),
        ],
        out_specs=[
            pl.BlockSpec((ke, _D), lambda i: (i, 0)),
            pl.BlockSpec((ke, 16), lambda i: (i, 0)),
        ],
        out_shape=[
            jax.ShapeDtypeStruct((_E, _D), _f32),
            jax.ShapeDtypeStruct((_E, 16), _f32),
        ],
    )


_CHS = 80
_NKS = _EPW // _CHS


def _scatter_body(row_hbm, ef0, trf0, zh,
                  aggp0, acp0,
                  ix0, ix1, rv0, rv1, tv0, tv1, big, sh_acc,
                  si0, si1, sd0, sd1):
    c = lax.axis_index("c")
    s = lax.axis_index("s")
    wid = s * _NC + c
    wbase = wid * _EPW
    IX = (ix0, ix1)
    RV = (rv0, rv1)
    TV = (tv0, tv1)
    SI = (si0, si1)
    SD = (sd0, sd1)
    z16 = jnp.zeros((16,), _f32)
    pltpu.sync_copy(zh, big)
    for ef, trf, aggp, acp in ((ef0, trf0, aggp0, acp0),):
        def clr(e, carry):
            big[e, pl.ds(0, 16)] = z16
            return carry
        lax.fori_loop(0, _CHS, clr, 0)

        for pass_b in (False, True):
            for i in range(_TSIZE // _CH):
                pltpu.sync_copy(big,
                                sh_acc.at[pl.ds(s * _TSTART + i * _CH, _CH)])
            plsc.subcore_barrier()

            def issue(k, p):
                @pl.when(k < _NKS)
                def _():
                    base = wbase + k * _CHS
                    pltpu.async_copy(row_hbm.at[pl.ds(base, _CHS)],
                                     IX[p], SI[p])
                    if pass_b:
                        pltpu.async_copy(trf.at[pl.ds(base * 16, _CHS * 16)],
                                         TV[p], SD[p])
                    else:
                        pltpu.async_copy(ef.at[pl.ds(base, _CHS)],
                                         RV[p], SD[p])

            def drain(k, p):
                @pl.when(k < _NKS)
                def _():
                    base = wbase + k * _CHS
                    pltpu.make_async_copy(row_hbm.at[pl.ds(base, _CHS)],
                                          IX[p], SI[p]).wait()
                    if pass_b:
                        pltpu.make_async_copy(
                            trf.at[pl.ds(base * 16, _CHS * 16)],
                            TV[p], SD[p]).wait()

                        def expand(e, carry2):
                            big[e, pl.ds(0, 16)] = TV[p][pl.ds(e * 16, 16)]
                            return carry2
                        lax.fori_loop(0, _CHS, expand, 0)
                        pltpu.sync_copy(big.at[pl.ds(0, _CHS)],
                                        sh_acc.at[IX[p]], add=True)
                    else:
                        pltpu.make_async_copy(ef.at[pl.ds(base, _CHS)],
                                              RV[p], SD[p]).wait()
                        pltpu.sync_copy(RV[p], sh_acc.at[IX[p]], add=True)

            issue(0, 0)

            def pipe(j2, carry):
                issue(2 * j2 + 1, 1)
                drain(2 * j2, 0)
                issue(2 * j2 + 2, 0)
                drain(2 * j2 + 1, 1)
                return carry
            lax.fori_loop(0, (_NKS + 1) // 2, pipe, 0)
            plsc.subcore_barrier()
            out = acp if pass_b else aggp
            pltpu.sync_copy(sh_acc.at[pl.ds(s * _TSTART, _TSIZE)],
                            out.at[pl.ds(c * _N + s * _TSTART, _TSIZE)])
            plsc.subcore_barrier()
            if pass_b:
                pass


def _node_body(h_ref, c16_ref, ap0_ref, ap1_ref, ac0_ref, ac1_ref,
               wnh_ref, wna_ref, bn1_ref, wn2_ref, bn2_ref, oh_ref, sel_ref,
               hout_ref, cout_ref):
    agg = ap0_ref[...] + ap1_ref[...]
    ac = ac0_ref[...] + ac1_ref[...]
    deg = jnp.sum(ac * oh_ref[...], axis=1, keepdims=True)
    ac16 = jnp.dot(ac, sel_ref[...], preferred_element_type=_f32)
    cout_ref[...] = c16_ref[...] + ac16 * (1.0 / jnp.maximum(deg, 1.0))
    hb = h_ref[...]
    t = jnp.maximum(
        jnp.dot(hb, wnh_ref[...], preferred_element_type=_f32)
        + jnp.dot(agg, wna_ref[...], preferred_element_type=_f32)
        + bn1_ref[...], 0.0)
    hout_ref[...] = (hb + jnp.dot(t, wn2_ref[...], preferred_element_type=_f32)
                     + bn2_ref[...])


def _node_call_kwargs():
    kn = 2000
    return dict(
        grid=(_N // kn,),
        in_specs=[
            pl.BlockSpec((kn, _D), lambda i: (i, 0)),
            pl.BlockSpec((kn, 16), lambda i: (i, 0)),
            pl.BlockSpec((kn, _D), lambda i: (i, 0)),
            pl.BlockSpec((kn, _D), lambda i: (i, 0)),
            pl.BlockSpec((kn, _D), lambda i: (i, 0)),
            pl.BlockSpec((kn, _D), lambda i: (i, 0)),
            pl.BlockSpec((_D, _H), lambda i: (0, 0)),
            pl.BlockSpec((_H, _H), lambda i: (0, 0)),
            pl.BlockSpec((1, _H), lambda i: (0, 0)),
            pl.BlockSpec((_H, _D), lambda i: (0, 0)),
            pl.BlockSpec((1, _D), lambda i: (0, 0)),
            pl.BlockSpec((1, _D), lambda i: (0, 0)),
            pl.BlockSpec((_D, 16), lambda i: (0, 0)),
        ],
        out_specs=[
            pl.BlockSpec((kn, _D), lambda i: (i, 0)),
            pl.BlockSpec((kn, 16), lambda i: (i, 0)),
        ],
        out_shape=[
            jax.ShapeDtypeStruct((_N, _D), _f32),
            jax.ShapeDtypeStruct((_N, 16), _f32),
        ],
    )


def kernel(h, coord, edge_index, edge_attr, We1, be1, We2, be2,
           Wn1, bn1, Wn2, bn2, Wc1, bc1, Wc2):
    row = edge_index[0]
    col = edge_index[1]
    hflat = h.reshape(_B * _N, _D)
    c12 = coord.reshape(_B * _N, 12)

    ws = We1[:_D]
    wt = We1[_D:2 * _D]
    wr = We1[2 * _D:2 * _D + 16]
    wa = We1[2 * _D + 16:]
    wcg = Wc2 @ jnp.asarray(_G4)
    g1 = jnp.asarray(_G1)
    g2 = jnp.asarray(_G2)
    g3 = jnp.asarray(_G3)
    p12 = jnp.asarray(_P12)
    p128 = jnp.asarray(_P128)
    oh12 = jnp.asarray(_OH12)
    oh128 = jnp.asarray(_OH128)
    sel = jnp.asarray(_SEL)
    be1r = be1.reshape(1, _H)
    be2r = be2.reshape(1, _H)
    bc1r = bc1.reshape(1, _H)
    bn1r = bn1.reshape(1, _H)
    bn2r = bn2.reshape(1, _D)
    wnh = Wn1[:_D]
    wna = Wn1[_D:]

    hws, hwt, c16, ct = pl.pallas_call(_pre_body, **_pre_call_kwargs())(
        hflat, c12, ws, wt, p12, p128)

    gather = pl.kernel(
        _gather_body,
        out_type=(
            jax.ShapeDtypeStruct((_E, _D), _f32),
            jax.ShapeDtypeStruct((_E * 16,), _f32),
        ),
        mesh=plsc.VectorSubcoreMesh(core_axis_name="c", subcore_axis_name="s"),
        scratch_types=(
            [pltpu.VMEM((_EPW,), jnp.int32)] * 2
            + [pltpu.VMEM((_CHG, _D), _f32)] * 8
            + [pltpu.VMEM((_CHG * 16,), _f32)] * 2
            + [pltpu.SemaphoreType.DMA] * 8
        ),
    )
    pre0, cdf0 = gather(row, col, hws[:_N], hwt[:_N], ct[:_N])
    pre1, cdf1 = gather(row, col, hws[_N:], hwt[_N:], ct[_N:])
    cd0 = cdf0.reshape(_E, 16)
    cd1 = cdf1.reshape(_E, 16)

    edge_call = pl.pallas_call(_edge_body, **_edge_call_kwargs())
    ef0, tr0 = edge_call(pre0, cd0, edge_attr, wr, wa, be1r,
                         We2, be2r, Wc1, bc1r, wcg, g1, g2, g3, oh12)
    ef1, tr1 = edge_call(pre1, cd1, edge_attr, wr, wa, be1r,
                         We2, be2r, Wc1, bc1r, wcg, g1, g2, g3, oh12)

    zh = jnp.zeros((_CH, _D), _f32)
    scatter = pl.kernel(
        _scatter_body,
        out_type=tuple(
            jax.ShapeDtypeStruct((_NC * _N, _D), _f32) for _ in range(2)),
        mesh=plsc.VectorSubcoreMesh(core_axis_name="c", subcore_axis_name="s"),
        scratch_types=(
            [pltpu.VMEM((_CHS,), jnp.int32)] * 2
            + [pltpu.VMEM((_CHS, _D), _f32)] * 2
            + [pltpu.VMEM((_CHS * 16,), _f32)] * 2
            + [pltpu.VMEM((_CH, _D), _f32),
               pltpu.VMEM_SHARED((_N, _D), _f32)]
            + [pltpu.SemaphoreType.DMA] * 4
        ),
    )
    aggp0, acp0 = scatter(row, ef0, tr0.reshape(-1), zh)
    aggp1, acp1 = scatter(row, ef1, tr1.reshape(-1), zh)
    node_call = pl.pallas_call(_node_body, **_node_call_kwargs())
    h0, c0 = node_call(hflat[:_N], c16[:_N], aggp0[:_N], aggp0[_N:],
                       acp0[:_N], acp0[_N:], wnh, wna, bn1r, Wn2, bn2r,
                       oh128, sel)
    h1, c1 = node_call(hflat[_N:], c16[_N:], aggp1[:_N], aggp1[_N:],
                       acp1[:_N], acp1[_N:], wnh, wna, bn1r, Wn2, bn2r,
                       oh128, sel)

    hs = jnp.stack([h0, h1])
    coords = jnp.stack([c0[:, :12].reshape(_N, 4, 3),
                        c1[:, :12].reshape(_N, 4, 3)])
    return (hs, coords)
